# baseline (speedup 1.0000x reference)
import jax
import jax.numpy as jnp
from jax import lax
from jax.experimental import pallas as pl
from jax.experimental.pallas import tpu as pltpu
from jax.experimental.pallas import tpu_sc as plsc

SPEAKER_SIZE = 100000
CHANNEL = 256
BATCH = 16384

_NC = 2
_NS = 16
_NW = _NC * _NS
_CHUNK = 32
_PER_W = BATCH // _NW
_NCHUNK = _PER_W // _CHUNK
_NBUF = 15


def _gather_kernel(table_hbm, idx_hbm, out_hbm, idx_v, rows_v, *sems):
    wid = lax.axis_index("s") * _NC + lax.axis_index("c")
    base = wid * _PER_W
    gsem = sems[:_NBUF]
    wsem = sems[_NBUF:]

    pltpu.sync_copy(idx_hbm.at[pl.ds(base, _PER_W)], idx_v)

    def start_gather(c):
        b = c % _NBUF
        return pltpu.async_copy(
            table_hbm.at[idx_v.at[pl.ds(c * _CHUNK, _CHUNK)]], rows_v.at[b],
            gsem[b])

    def start_write(c):
        b = c % _NBUF
        return pltpu.async_copy(rows_v.at[b],
                                out_hbm.at[pl.ds(base + c * _CHUNK, _CHUNK)],
                                wsem[b])

    g = [start_gather(c) for c in range(min(_NBUF, _NCHUNK))]
    g += [None] * (_NCHUNK - len(g))
    w = [None] * _NCHUNK
    for c in range(_NCHUNK):
        nxt = c + _NBUF
        g[c].wait()
        w[c] = start_write(c)
        if nxt < _NCHUNK:
            w[nxt - _NBUF].wait()
            g[nxt] = start_gather(nxt)
    for c in range(max(0, _NCHUNK - _NBUF), _NCHUNK):
        if w[c] is not None:
            w[c].wait()


@jax.jit
def kernel(spk_id, table):
    run = pl.kernel(
        _gather_kernel,
        out_type=jax.ShapeDtypeStruct((BATCH, CHANNEL), jnp.float32),
        mesh=plsc.VectorSubcoreMesh(core_axis_name="c", subcore_axis_name="s"),
        scratch_types=(
            [pltpu.VMEM((_PER_W,), jnp.int32),
             pltpu.VMEM((_NBUF, _CHUNK, CHANNEL), jnp.float32)]
            + [pltpu.SemaphoreType.DMA] * (2 * _NBUF)
        ),
    )
    return run(table, spk_id)
